# baseline (speedup 1.0000x reference)
import functools

import jax
import jax.numpy as jnp
from jax import lax
from jax.experimental import pallas as pl
from jax.experimental.pallas import tpu as pltpu

_D = _H = _W = 128
_YC = 8
_NT = _H // _YC


def _tile_kernel(bnd_ref, fwd_ref, bwd_ref, out_ref):
    z = pl.program_id(1)
    zf = z.astype(jnp.float32)
    iota_s = lax.broadcasted_iota(jnp.int32, (_YC, _W), 0).astype(jnp.float32)
    iota_l = lax.broadcasted_iota(jnp.int32, (_YC, _W), 1).astype(jnp.float32)

    def tile_body(yc, tot):
        ys = pl.multiple_of(yc * _YC, _YC)
        fz = fwd_ref[0, 0, 0, pl.ds(ys, _YC), :]
        fy = fwd_ref[0, 1, 0, pl.ds(ys, _YC), :]
        fx = fwd_ref[0, 2, 0, pl.ds(ys, _YC), :]

        cz = jnp.clip(zf + fz, -2.0, 129.0)
        cy = jnp.clip(iota_s + (yc * _YC).astype(jnp.float32) + fy, -2.0, 129.0)
        cx = jnp.clip(iota_l + fx, -2.0, 129.0)
        z0f = jnp.floor(cz)
        y0f = jnp.floor(cy)
        x0f = jnp.floor(cx)
        wz = cz - z0f
        wy = cy - y0f
        wx = cx - x0f
        z0 = z0f.astype(jnp.int32)
        y0 = y0f.astype(jnp.int32)
        x0 = x0f.astype(jnp.int32)

        x1 = x0 + 1
        wx0 = jnp.where((x0 >= 0) & (x0 < _W), 1.0 - wx, 0.0)
        wx1 = jnp.where((x1 >= 0) & (x1 < _W), wx, 0.0)
        x0c = jnp.clip(x0, 0, _W - 1)
        x1c = jnp.clip(x1, 0, _W - 1)

        zlo = bnd_ref[0, 0, yc, 0]
        zhi = bnd_ref[0, 0, yc, 1]
        klo = bnd_ref[0, 0, yc, 2]
        khi = bnd_ref[0, 0, yc, 3]

        zero = jnp.zeros((_YC, _W), jnp.float32)

        def chunk_body(k, acc):
            base = pl.multiple_of(k * 8, 8)
            d0 = y0 - base
            d1 = d0 + 1
            wy0 = jnp.where((d0 >= 0) & (d0 < 8), 1.0 - wy, 0.0)
            wy1 = jnp.where((d1 >= 0) & (d1 < 8), wy, 0.0)
            i0 = jnp.clip(d0, 0, 7)
            i1 = jnp.clip(d1, 0, 7)

            def plane_body(zi, acc):
                wzeff = (jnp.where(z0 == zi, 1.0 - wz, 0.0)
                         + jnp.where(z0 == zi - 1, wz, 0.0))
                out = []
                for c in range(3):
                    s = bwd_ref[0, c, zi, pl.ds(base, 8), :]
                    g0 = jnp.take_along_axis(s, i0, axis=0)
                    g1 = jnp.take_along_axis(s, i1, axis=0)
                    row = g0 * wy0 + g1 * wy1
                    v0 = jnp.take_along_axis(row, x0c, axis=1)
                    v1 = jnp.take_along_axis(row, x1c, axis=1)
                    out.append(acc[c] + wzeff * (v0 * wx0 + v1 * wx1))
                return tuple(out)

            return lax.fori_loop(zlo, zhi + 1, plane_body, acc)

        acc = lax.fori_loop(klo, khi + 1, chunk_body, (zero, zero, zero))
        sq = ((fz + acc[0]) ** 2 + (fy + acc[1]) ** 2 + (fx + acc[2]) ** 2)
        return tot + sq

    total = lax.fori_loop(0, _NT, tile_body, jnp.zeros((_YC, _W), jnp.float32))

    @pl.when(z == 0)
    def _():
        out_ref[0] = total

    @pl.when(z != 0)
    def _():
        out_ref[0] += total


@functools.partial(jax.jit, static_argnames=("interpret",))
def kernel(forward_disp, backward_disp, interpret=False):
    B = forward_disp.shape[0]

    zidx = lax.broadcasted_iota(jnp.float32, (1, _D, 1, 1, 1), 1)
    yidx = lax.broadcasted_iota(jnp.float32, (1, 1, 1, _YC, 1), 3)
    ycidx = lax.broadcasted_iota(jnp.float32, (1, 1, _NT, 1, 1), 2)
    fz = forward_disp[:, 0].reshape(B, _D, _NT, _YC, _W)
    fy = forward_disp[:, 1].reshape(B, _D, _NT, _YC, _W)
    z0 = jnp.floor(jnp.clip(fz + zidx, -2.0, 129.0)).astype(jnp.int32)
    y0 = jnp.floor(jnp.clip(fy + yidx + ycidx * _YC, -2.0, 129.0)).astype(jnp.int32)
    zlo = jnp.clip(jnp.min(z0, axis=(3, 4)), 0, _D - 1)
    zhi = jnp.clip(jnp.max(z0, axis=(3, 4)) + 1, 0, _D - 1)
    rlo = jnp.clip(jnp.min(y0, axis=(3, 4)), 0, _H - 1) >> 3
    rhi = (jnp.clip(jnp.max(y0, axis=(3, 4)) + 1, 0, _H - 1)) >> 3
    bounds = jnp.stack([zlo, zhi, rlo, rhi], axis=-1)

    out = pl.pallas_call(
        _tile_kernel,
        grid=(B, _D),
        in_specs=[
            pl.BlockSpec((1, 1, _NT, 4), lambda b, z: (b, z, 0, 0),
                         memory_space=pltpu.SMEM),
            pl.BlockSpec((1, 3, 1, _H, _W), lambda b, z: (b, 0, z, 0, 0)),
            pl.BlockSpec((1, 3, _D, _H, _W), lambda b, z: (b, 0, 0, 0, 0)),
        ],
        out_specs=pl.BlockSpec((1, _YC, _W), lambda b, z: (b, 0, 0)),
        out_shape=jax.ShapeDtypeStruct((B, _YC, _W), jnp.float32),
        compiler_params=pltpu.CompilerParams(
            dimension_semantics=("parallel", "arbitrary")),
        interpret=interpret,
    )(bounds, forward_disp, backward_disp)

    n = forward_disp.size
    loss = jnp.sum(out) / jnp.float32(n)
    return jnp.nan_to_num(loss, nan=0.0, posinf=1000.0, neginf=0.0)


# speedup vs baseline: 2.6175x; 2.6175x over previous
import functools

import jax
import jax.numpy as jnp
from jax import lax
from jax.experimental import pallas as pl
from jax.experimental.pallas import tpu as pltpu

_D = _H = _W = 128
_YC = 8
_NT = _H // _YC
_ZU = 4


def _tile_kernel(bnd_ref, fwd_ref, bwd_ref, out_ref):
    z = pl.program_id(1)
    zf = z.astype(jnp.float32)
    iota_s = lax.broadcasted_iota(jnp.int32, (_YC, _W), 0).astype(jnp.float32)
    iota_l = lax.broadcasted_iota(jnp.int32, (_YC, _W), 1).astype(jnp.float32)

    def tile_body(yc, tot):
        ys = pl.multiple_of(yc * _YC, _YC)
        fz = fwd_ref[0, 0, 0, pl.ds(ys, _YC), :]
        fy = fwd_ref[0, 1, 0, pl.ds(ys, _YC), :]
        fx = fwd_ref[0, 2, 0, pl.ds(ys, _YC), :]

        cz = jnp.clip(zf + fz, -2.0, 129.0)
        cy = jnp.clip(iota_s + (yc * _YC).astype(jnp.float32) + fy, -2.0, 129.0)
        cx = jnp.clip(iota_l + fx, -2.0, 129.0)
        z0f = jnp.floor(cz)
        y0f = jnp.floor(cy)
        x0f = jnp.floor(cx)
        wz = cz - z0f
        wy = cy - y0f
        wx = cx - x0f
        z0 = z0f.astype(jnp.int32)
        y0 = y0f.astype(jnp.int32)
        x0 = x0f.astype(jnp.int32)

        x1 = x0 + 1
        wx0 = jnp.where((x0 >= 0) & (x0 < _W), 1.0 - wx, 0.0)
        wx1 = jnp.where((x1 >= 0) & (x1 < _W), wx, 0.0)
        x0c = jnp.clip(x0, 0, _W - 1)
        x1c = jnp.clip(x1, 0, _W - 1)

        wzA = jnp.where(z0 <= _D - 1, 1.0 - wz, 0.0)
        wzB = jnp.where(z0 <= _D - 2, wz, 0.0)

        zlo = bnd_ref[0, 0, yc, 0]
        ng = bnd_ref[0, 0, yc, 1]
        klo = bnd_ref[0, 0, yc, 2]
        khi = bnd_ref[0, 0, yc, 3]

        zero = jnp.zeros((_YC, _W), jnp.float32)

        def chunk_body(k, acc):
            base = pl.multiple_of(k * 8, 8)
            d0 = y0 - base
            d1 = d0 + 1
            wy0 = jnp.where((d0 >= 0) & (d0 < 8), 1.0 - wy, 0.0)
            wy1 = jnp.where((d1 >= 0) & (d1 < 8), wy, 0.0)
            i0 = jnp.clip(d0, 0, 7)
            i1 = jnp.clip(d1, 0, 7)

            def plane_group_body(j, acc):
                zi0 = zlo + j * _ZU
                vals = []
                for u in range(_ZU):
                    zi = zi0 + u
                    zil = jnp.minimum(zi, _D - 1)
                    wzeff = (jnp.where(z0 == zi, wzA, 0.0)
                             + jnp.where(z0 == zi - 1, wzB, 0.0))
                    for c in range(3):
                        s = bwd_ref[0, c, zil, pl.ds(base, 8), :]
                        g0 = jnp.take_along_axis(s, i0, axis=0)
                        g1 = jnp.take_along_axis(s, i1, axis=0)
                        row = g0 * wy0 + g1 * wy1
                        v0 = jnp.take_along_axis(row, x0c, axis=1)
                        v1 = jnp.take_along_axis(row, x1c, axis=1)
                        vals.append((c, wzeff * (v0 * wx0 + v1 * wx1)))
                out = list(acc)
                for c, v in vals:
                    out[c] = out[c] + v
                return tuple(out)

            return lax.fori_loop(0, ng, plane_group_body, acc)

        acc = lax.fori_loop(klo, khi + 1, chunk_body, (zero, zero, zero))
        sq = ((fz + acc[0]) ** 2 + (fy + acc[1]) ** 2 + (fx + acc[2]) ** 2)
        return tot + sq

    total = lax.fori_loop(0, _NT, tile_body, jnp.zeros((_YC, _W), jnp.float32))

    @pl.when(z == 0)
    def _():
        out_ref[0] = total

    @pl.when(z != 0)
    def _():
        out_ref[0] += total


@functools.partial(jax.jit, static_argnames=("interpret",))
def kernel(forward_disp, backward_disp, interpret=False):
    B = forward_disp.shape[0]

    zidx = lax.broadcasted_iota(jnp.float32, (1, _D, 1, 1, 1), 1)
    yidx = lax.broadcasted_iota(jnp.float32, (1, 1, 1, _YC, 1), 3)
    ycidx = lax.broadcasted_iota(jnp.float32, (1, 1, _NT, 1, 1), 2)
    fz = forward_disp[:, 0].reshape(B, _D, _NT, _YC, _W)
    fy = forward_disp[:, 1].reshape(B, _D, _NT, _YC, _W)
    z0 = jnp.floor(jnp.clip(fz + zidx, -2.0, 129.0)).astype(jnp.int32)
    y0 = jnp.floor(jnp.clip(fy + yidx + ycidx * _YC, -2.0, 129.0)).astype(jnp.int32)
    zlo = jnp.clip(jnp.min(z0, axis=(3, 4)), 0, _D - 1)
    zhi = jnp.clip(jnp.max(z0, axis=(3, 4)) + 1, 0, _D - 1)
    ngrp = (zhi - zlo + _ZU) // _ZU
    rlo = jnp.clip(jnp.min(y0, axis=(3, 4)), 0, _H - 1) >> 3
    rhi = (jnp.clip(jnp.max(y0, axis=(3, 4)) + 1, 0, _H - 1)) >> 3
    bounds = jnp.stack([zlo, ngrp, rlo, rhi], axis=-1)

    out = pl.pallas_call(
        _tile_kernel,
        grid=(B, _D),
        in_specs=[
            pl.BlockSpec((1, 1, _NT, 4), lambda b, z: (b, z, 0, 0),
                         memory_space=pltpu.SMEM),
            pl.BlockSpec((1, 3, 1, _H, _W), lambda b, z: (b, 0, z, 0, 0)),
            pl.BlockSpec((1, 3, _D, _H, _W), lambda b, z: (b, 0, 0, 0, 0)),
        ],
        out_specs=pl.BlockSpec((1, _YC, _W), lambda b, z: (b, 0, 0)),
        out_shape=jax.ShapeDtypeStruct((B, _YC, _W), jnp.float32),
        compiler_params=pltpu.CompilerParams(
            dimension_semantics=("parallel", "arbitrary")),
        interpret=interpret,
    )(bounds, forward_disp, backward_disp)

    n = forward_disp.size
    loss = jnp.sum(out) / jnp.float32(n)
    return jnp.nan_to_num(loss, nan=0.0, posinf=1000.0, neginf=0.0)
